# baseline (speedup 1.0000x reference)
import functools

import jax
import jax.numpy as jnp
from jax import lax
from jax.experimental import pallas as pl
from jax.experimental.pallas import tpu as pltpu
from jax.experimental.pallas import tpu_sc as plsc

B = 16384
GEO1 = 1000
D1, D2, D3 = 16, 128, 128
D = 128
LATENT = 16
CH = 128

_NC, _NS = 2, 16
_NW = _NC * _NS


def _sc_gather(i1, i2, i3, emb1, emb2, emb3):
    bh = i1.shape[0]
    cpw = (bh // CH) // _NW
    bpw = cpw * CH
    ng = bpw // 16
    slab_w = bh // 8
    nhalf = cpw // 2
    mesh = plsc.VectorSubcoreMesh(core_axis_name="c", subcore_axis_name="s")

    @functools.partial(
        pl.kernel,
        out_type=(
            jax.ShapeDtypeStruct((8 * D1, slab_w), jnp.float32),
            jax.ShapeDtypeStruct((bh, D), jnp.float32),
            jax.ShapeDtypeStruct((bh, D), jnp.float32),
        ),
        mesh=mesh,
        compiler_params=pltpu.CompilerParams(use_tc_tiling_on_sc=False,
                                             needs_layout_passes=False),
        scratch_types=[
            pltpu.VMEM((bpw,), jnp.int32),
            pltpu.VMEM((cpw, CH), jnp.int32),
            pltpu.VMEM((cpw, CH), jnp.int32),
            pltpu.VMEM((2 * CH, D), jnp.float32),
            pltpu.VMEM((2 * CH, D), jnp.float32),
            pltpu.VMEM((GEO1 * D1,), jnp.float32),
            pltpu.VMEM((D1, bpw), jnp.float32),
            pltpu.SemaphoreType.DMA,
            pltpu.SemaphoreType.DMA,
            pltpu.SemaphoreType.DMA,
            pltpu.SemaphoreType.DMA,
        ],
    )
    def k(i1r, i2r, i3r, e1r, e2r, e3r, g1r, g2r, g3r,
          idx1, idx2, idx3, buf_a, buf_b, e1v, g1tb,
          sem_i, sem_g, sem_o, sem_e):
        c = lax.axis_index("c")
        s = lax.axis_index("s")
        wid = s * _NC + c
        base = wid * cpw
        row_base = wid * bpw

        e1cp = pltpu.async_copy(e1r, e1v, sem_e)
        icps = [pltpu.async_copy(i1r.at[pl.ds(row_base, bpw)], idx1, sem_i),
                pltpu.async_copy(i2r.at[pl.ds(base, cpw)], idx2, sem_i),
                pltpu.async_copy(i3r.at[pl.ds(base, cpw)], idx3, sem_i)]
        for cp in icps:
            cp.wait()

        units = [(er, idx, gr, h)
                 for er, idx, gr in ((e2r, idx2, g2r), (e3r, idx3, g3r))
                 for h in range(nhalf)]
        nu = len(units)
        bufs = (buf_a, buf_b)

        def fire_gather(u):
            er, idx, _, h = units[u]
            return [pltpu.async_copy(er.at[idx.at[2 * h + j]],
                                     bufs[u % 2].at[pl.ds(j * CH, CH)],
                                     sem_g)
                    for j in range(2)]

        def fire_out(u):
            _, _, gr, h = units[u]
            return pltpu.async_copy(
                bufs[u % 2],
                gr.at[pl.ds(row_base + 2 * h * CH, 2 * CH)], sem_o)

        gcps = {0: fire_gather(0), 1: fire_gather(1)}

        e1cp.wait()

        for g in range(ng):
            iv16 = idx1[pl.ds(g * 16, 16)] * D1
            for f in range(D1):
                vals = plsc.load_gather(e1v, [iv16 + f])
                g1tb[f, pl.ds(g * 16, 16)] = vals
        slab = row_base // slab_w
        col0 = row_base % slab_w
        g1cp = pltpu.async_copy(
            g1tb,
            g1r.at[pl.ds(slab * D1, D1), pl.ds(col0, bpw)], sem_e)

        ocps = {}
        for u in range(nu):
            for cp in gcps[u]:
                cp.wait()
            ocps[u] = fire_out(u)
            if u + 2 < nu:
                ocps[u].wait()
                gcps[u + 2] = fire_gather(u + 2)
        ocps[nu - 2].wait()
        ocps[nu - 1].wait()
        g1cp.wait()

    return k(i1, i2, i3, emb1, emb2, emb3)


def _tc_body(g1t, g2, g3, w1, w2, w3, bias, out):
    h1t = jnp.maximum(g1t[...], 0.0)
    h2 = jnp.maximum(g2[...], 0.0)
    h3 = jnp.maximum(g3[...], 0.0)
    acc = lax.dot_general(w1[...], h1t, (((0,), (0,)), ((), ())),
                          preferred_element_type=jnp.float32)
    acc = acc + lax.dot_general(w2[...], h2, (((0,), (1,)), ((), ())),
                                preferred_element_type=jnp.float32)
    acc = acc + lax.dot_general(w3[...], h3, (((0,), (1,)), ((), ())),
                                preferred_element_type=jnp.float32)
    out[...] = jnp.maximum(acc + bias[...], 0.0)


def _tc_linear(g1p, g2, g3, w1, w2, w3, bias):
    bh = g2.shape[0]
    R = bh // 8
    grid = (bh // R,)
    return pl.pallas_call(
        _tc_body,
        grid=grid,
        in_specs=[
            pl.BlockSpec((D1, R), lambda i: (i, 0)),
            pl.BlockSpec((R, D), lambda i: (i, 0)),
            pl.BlockSpec((R, D), lambda i: (i, 0)),
            pl.BlockSpec((D1, LATENT), lambda i: (0, 0)),
            pl.BlockSpec((D, LATENT), lambda i: (0, 0)),
            pl.BlockSpec((D, LATENT), lambda i: (0, 0)),
            pl.BlockSpec((LATENT, 1), lambda i: (0, 0)),
        ],
        out_specs=pl.BlockSpec((LATENT, R), lambda i: (0, i)),
        out_shape=jax.ShapeDtypeStruct((LATENT, bh), jnp.float32),
    )(g1p, g2, g3, w1, w2, w3, bias)


def kernel(x, emb1, emb2, emb3, W, b):
    xi = x.astype(jnp.int32)
    i1 = xi[:, 0]
    i2 = xi[:, 1]
    i3 = xi[:, 2]
    e1f = emb1.reshape(GEO1 * D1)

    w1 = W[:D1]
    w2 = W[D1:D1 + D2]
    w3 = W[D1 + D2:]
    bias = b.reshape(LATENT, 1)

    H = B // 2
    outs = []
    for hh in range(2):
        sl = slice(hh * H, (hh + 1) * H)
        g1p, g2, g3 = _sc_gather(i1[sl],
                                 i2[sl].reshape(H // CH, CH),
                                 i3[sl].reshape(H // CH, CH),
                                 e1f, emb2, emb3)
        outs.append(_tc_linear(g1p, g2, g3, w1, w2, w3, bias))
    return jnp.concatenate(outs, axis=1).T


# speedup vs baseline: 1.3530x; 1.3530x over previous
import functools

import jax
import jax.numpy as jnp
from jax import lax
from jax.experimental import pallas as pl
from jax.experimental.pallas import tpu as pltpu
from jax.experimental.pallas import tpu_sc as plsc

B = 16384
GEO1 = 1000
D1, D2, D3 = 16, 128, 128
D = 128
LATENT = 16
CH = 128
NCHUNK = B // CH

_NC, _NS = 2, 16
_NW = _NC * _NS
_CPW = NCHUNK // _NW
_BPW = _CPW * CH
_NG = _BPW // 16


def _sc_gather(i1, i2, i3, emb1, emb2, emb3):
    mesh = plsc.VectorSubcoreMesh(core_axis_name="c", subcore_axis_name="s")

    @functools.partial(
        pl.kernel,
        out_type=(
            jax.ShapeDtypeStruct((8 * D1, B // 8), jnp.float32),
            jax.ShapeDtypeStruct((B, D), jnp.float32),
            jax.ShapeDtypeStruct((B, D), jnp.float32),
        ),
        mesh=mesh,
        compiler_params=pltpu.CompilerParams(use_tc_tiling_on_sc=False,
                                             needs_layout_passes=False),
        scratch_types=[
            pltpu.VMEM((_BPW,), jnp.int32),
            pltpu.VMEM((_CPW, CH), jnp.int32),
            pltpu.VMEM((_CPW, CH), jnp.int32),
            pltpu.VMEM((CH, D), jnp.float32),
            pltpu.VMEM((CH, D), jnp.float32),
            pltpu.VMEM((CH, D), jnp.float32),
            pltpu.VMEM((CH, D), jnp.float32),
            pltpu.VMEM((CH, D), jnp.float32),
            pltpu.VMEM((CH, D), jnp.float32),
            pltpu.VMEM((GEO1 * D1,), jnp.float32),
            pltpu.VMEM((D1, _BPW), jnp.float32),
            pltpu.SemaphoreType.DMA,
            pltpu.SemaphoreType.DMA,
            pltpu.SemaphoreType.DMA,
            pltpu.SemaphoreType.DMA,
        ],
    )
    def k(i1r, i2r, i3r, e1r, e2r, e3r, g1r, g2r, g3r,
          idx1, idx2, idx3, buf_a, buf_b, buf_c, buf_d, buf_e, buf_f,
          e1v, g1tb, sem_i, sem_g, sem_o, sem_e):
        c = lax.axis_index("c")
        s = lax.axis_index("s")
        wid = s * _NC + c
        base = wid * _CPW
        row_base = wid * _BPW

        e1cp = pltpu.async_copy(e1r, e1v, sem_e)
        icps = [pltpu.async_copy(i1r.at[pl.ds(row_base, _BPW)], idx1, sem_i),
                pltpu.async_copy(i2r.at[pl.ds(base, _CPW)], idx2, sem_i),
                pltpu.async_copy(i3r.at[pl.ds(base, _CPW)], idx3, sem_i)]
        for cp in icps:
            cp.wait()

        units = [(er, idx, gr, h)
                 for er, idx, gr in ((e2r, idx2, g2r), (e3r, idx3, g3r))
                 for h in range(_CPW)]
        bufs = (buf_a, buf_b, buf_c, buf_d, buf_e, buf_f)
        nbuf = len(bufs)

        def fire_gather(u):
            er, idx, _, h = units[u]
            return pltpu.async_copy(er.at[idx.at[h]], bufs[u % nbuf], sem_g)

        def fire_out(u):
            _, _, gr, h = units[u]
            return pltpu.async_copy(
                bufs[u % nbuf],
                gr.at[pl.ds(row_base + h * CH, CH)], sem_o)

        gcps = {u: fire_gather(u) for u in range(nbuf)}

        e1cp.wait()

        for g in range(_NG):
            iv16 = idx1[pl.ds(g * 16, 16)] * D1
            for f in range(D1):
                vals = plsc.load_gather(e1v, [iv16 + f])
                g1tb[f, pl.ds(g * 16, 16)] = vals
        slab = row_base // (B // 8)
        col0 = row_base % (B // 8)
        g1cp = pltpu.async_copy(
            g1tb,
            g1r.at[pl.ds(slab * D1, D1), pl.ds(col0, _BPW)], sem_e)

        nu = len(units)
        ocps = {}
        waited = set()
        for u in range(nu):
            gcps[u].wait()
            ocps[u] = fire_out(u)
            if u + nbuf < nu:
                ocps[u].wait()
                waited.add(u)
                gcps[u + nbuf] = fire_gather(u + nbuf)
        for u in range(nu):
            if u not in waited:
                ocps[u].wait()
        g1cp.wait()

    return k(i1, i2, i3, emb1, emb2, emb3)


def _tc_body(g1t, g2, g3, w1, w2, w3, bias, out):
    h1t = jnp.maximum(g1t[...], 0.0)
    h2 = jnp.maximum(g2[...], 0.0)
    h3 = jnp.maximum(g3[...], 0.0)
    acc = lax.dot_general(w1[...], h1t, (((0,), (0,)), ((), ())),
                          preferred_element_type=jnp.float32)
    acc = acc + lax.dot_general(w2[...], h2, (((0,), (1,)), ((), ())),
                                preferred_element_type=jnp.float32)
    acc = acc + lax.dot_general(w3[...], h3, (((0,), (1,)), ((), ())),
                                preferred_element_type=jnp.float32)
    out[...] = jnp.maximum(acc + bias[...], 0.0)


def _tc_linear(g1t, g2, g3, w1, w2, w3, bias):
    R = 2048
    grid = (B // R,)
    return pl.pallas_call(
        _tc_body,
        grid=grid,
        in_specs=[
            pl.BlockSpec((D1, R), lambda i: (i, 0)),
            pl.BlockSpec((R, D), lambda i: (i, 0)),
            pl.BlockSpec((R, D), lambda i: (i, 0)),
            pl.BlockSpec((D1, LATENT), lambda i: (0, 0)),
            pl.BlockSpec((D, LATENT), lambda i: (0, 0)),
            pl.BlockSpec((D, LATENT), lambda i: (0, 0)),
            pl.BlockSpec((LATENT, 1), lambda i: (0, 0)),
        ],
        out_specs=pl.BlockSpec((LATENT, R), lambda i: (0, i)),
        out_shape=jax.ShapeDtypeStruct((LATENT, B), jnp.float32),
    )(g1t, g2, g3, w1, w2, w3, bias)


def kernel(x, emb1, emb2, emb3, W, b):
    xi = x.astype(jnp.int32)
    i1 = xi[:, 0]
    i2 = xi[:, 1].reshape(NCHUNK, CH)
    i3 = xi[:, 2].reshape(NCHUNK, CH)

    g1t, g2, g3 = _sc_gather(i1, i2, i3, emb1.reshape(GEO1 * D1), emb2, emb3)

    w1 = W[:D1]
    w2 = W[D1:D1 + D2]
    w3 = W[D1 + D2:]
    bias = b.reshape(LATENT, 1)
    return _tc_linear(g1t, g2, g3, w1, w2, w3, bias).T


# speedup vs baseline: 1.4106x; 1.0426x over previous
import functools

import jax
import jax.numpy as jnp
from jax import lax
from jax.experimental import pallas as pl
from jax.experimental.pallas import tpu as pltpu
from jax.experimental.pallas import tpu_sc as plsc

B = 16384
GEO1 = 1000
D1, D2, D3 = 16, 128, 128
D = 128
LATENT = 16
CH = 128
NCHUNK = B // CH

_NC, _NS = 2, 16
_NW = _NC * _NS
_CPW = NCHUNK // _NW
_BPW = _CPW * CH
_NG = _BPW // 16


def _sc_gather(i1, i2, i3, emb1, emb2, emb3):
    mesh = plsc.VectorSubcoreMesh(core_axis_name="c", subcore_axis_name="s")

    @functools.partial(
        pl.kernel,
        out_type=(
            jax.ShapeDtypeStruct((8 * D1, (B // 8) // D, D), jnp.float32),
            jax.ShapeDtypeStruct((B, D), jnp.float32),
            jax.ShapeDtypeStruct((B, D), jnp.float32),
        ),
        mesh=mesh,
        compiler_params=pltpu.CompilerParams(use_tc_tiling_on_sc=False,
                                             needs_layout_passes=False),
        scratch_types=[
            pltpu.VMEM((_BPW,), jnp.int32),
            pltpu.VMEM((_CPW, CH), jnp.int32),
            pltpu.VMEM((_CPW, CH), jnp.int32),
            pltpu.VMEM((CH, D), jnp.float32),
            pltpu.VMEM((CH, D), jnp.float32),
            pltpu.VMEM((CH, D), jnp.float32),
            pltpu.VMEM((CH, D), jnp.float32),
            pltpu.VMEM((CH, D), jnp.float32),
            pltpu.VMEM((CH, D), jnp.float32),
            pltpu.VMEM((GEO1 * D1,), jnp.float32),
            pltpu.VMEM((D1, _BPW // D, D), jnp.float32),
            pltpu.SemaphoreType.DMA,
            pltpu.SemaphoreType.DMA,
            pltpu.SemaphoreType.DMA,
            pltpu.SemaphoreType.DMA,
        ],
    )
    def k(i1r, i2r, i3r, e1r, e2r, e3r, g1r, g2r, g3r,
          idx1, idx2, idx3, buf_a, buf_b, buf_c, buf_d, buf_e, buf_f,
          e1v, g1tb, sem_i, sem_g, sem_o, sem_e):
        c = lax.axis_index("c")
        s = lax.axis_index("s")
        wid = s * _NC + c
        base = wid * _CPW
        row_base = wid * _BPW

        e1cp = pltpu.async_copy(e1r, e1v, sem_e)
        icps = [pltpu.async_copy(i1r.at[pl.ds(row_base, _BPW)], idx1, sem_i),
                pltpu.async_copy(i2r.at[pl.ds(base, _CPW)], idx2, sem_i),
                pltpu.async_copy(i3r.at[pl.ds(base, _CPW)], idx3, sem_i)]
        for cp in icps:
            cp.wait()

        units = [(er, idx, gr, h)
                 for er, idx, gr in ((e2r, idx2, g2r), (e3r, idx3, g3r))
                 for h in range(_CPW)]
        bufs = (buf_a, buf_b, buf_c, buf_d, buf_e, buf_f)
        nbuf = len(bufs)

        def fire_gather(u):
            er, idx, _, h = units[u]
            return pltpu.async_copy(er.at[idx.at[h]], bufs[u % nbuf], sem_g)

        def fire_out(u):
            _, _, gr, h = units[u]
            return pltpu.async_copy(
                bufs[u % nbuf],
                gr.at[pl.ds(row_base + h * CH, CH)], sem_o)

        gcps = {u: fire_gather(u) for u in range(nbuf)}

        e1cp.wait()

        for g in range(_NG):
            iv16 = idx1[pl.ds(g * 16, 16)] * D1
            for f in range(D1):
                vals = plsc.load_gather(e1v, [iv16 + f])
                g1tb[f, g // 8, pl.ds((g % 8) * 16, 16)] = vals
        slab = row_base // (B // 8)
        col0 = row_base % (B // 8)
        g1cp = pltpu.async_copy(
            g1tb,
            g1r.at[pl.ds(slab * D1, D1), pl.ds(col0 // D, _BPW // D)], sem_e)

        nu = len(units)
        ocps = {}
        waited = set()
        for u in range(nu):
            gcps[u].wait()
            ocps[u] = fire_out(u)
            if u + nbuf < nu:
                ocps[u].wait()
                waited.add(u)
                gcps[u + nbuf] = fire_gather(u + nbuf)
        for u in range(nu):
            if u not in waited:
                ocps[u].wait()
        g1cp.wait()

    return k(i1, i2, i3, emb1, emb2, emb3)


def _tc_body(g1b, g2, g3, w1, w2, w3, bias, out):
    h2 = jnp.maximum(g2[...], 0.0)
    h3 = jnp.maximum(g3[...], 0.0)
    acc = lax.dot_general(w2[...], h2, (((0,), (1,)), ((), ())),
                          preferred_element_type=jnp.float32)
    acc = acc + lax.dot_general(w3[...], h3, (((0,), (1,)), ((), ())),
                                preferred_element_type=jnp.float32)
    acc = acc + bias[...]
    for v in range(acc.shape[1] // D):
        h1v = jnp.maximum(g1b[:, v, :], 0.0)
        accv = lax.dot_general(w1[...], h1v, (((0,), (0,)), ((), ())),
                               preferred_element_type=jnp.float32)
        out[:, pl.ds(v * D, D)] = jnp.maximum(
            acc[:, v * D:(v + 1) * D] + accv, 0.0)


def _tc_linear(g1t, g2, g3, w1, w2, w3, bias):
    R = 2048
    grid = (B // R,)
    return pl.pallas_call(
        _tc_body,
        grid=grid,
        in_specs=[
            pl.BlockSpec((D1, R // D, D), lambda i: (i, 0, 0)),
            pl.BlockSpec((R, D), lambda i: (i, 0)),
            pl.BlockSpec((R, D), lambda i: (i, 0)),
            pl.BlockSpec((D1, LATENT), lambda i: (0, 0)),
            pl.BlockSpec((D, LATENT), lambda i: (0, 0)),
            pl.BlockSpec((D, LATENT), lambda i: (0, 0)),
            pl.BlockSpec((LATENT, 1), lambda i: (0, 0)),
        ],
        out_specs=pl.BlockSpec((LATENT, R), lambda i: (0, i)),
        out_shape=jax.ShapeDtypeStruct((LATENT, B), jnp.float32),
    )(g1t, g2, g3, w1, w2, w3, bias)


def kernel(x, emb1, emb2, emb3, W, b):
    xi = x.astype(jnp.int32)
    i1 = xi[:, 0]
    i2 = xi[:, 1].reshape(NCHUNK, CH)
    i3 = xi[:, 2].reshape(NCHUNK, CH)

    g1t, g2, g3 = _sc_gather(i1, i2, i3, emb1.reshape(GEO1 * D1), emb2, emb3)

    w1 = W[:D1]
    w2 = W[D1:D1 + D2]
    w3 = W[D1 + D2:]
    bias = b.reshape(LATENT, 1)
    return _tc_linear(g1t, g2, g3, w1, w2, w3, bias).T
